# baseline (speedup 1.0000x reference)
import functools

import jax
import jax.numpy as jnp
from jax import lax
from jax.experimental import pallas as pl
from jax.experimental.pallas import tpu as pltpu
from jax.experimental.pallas import tpu_sc as plsc

_N = 2048
_D = 1024
_E = 64
_I = 2048
_K = 2
_A = _N * _K
_T = 128
_NB = 96
_ND = _NB * _T

_SC_CORES = 2
_SC_SUBCORES = 16
_SC_WORKERS = _SC_CORES * _SC_SUBCORES
_SC_CHUNK = 64


def _router_top2(x_flat, router):
    blk = 256

    def body(x_ref, r_ref, idx_ref, w_ref):
        l = jnp.dot(x_ref[...], r_ref[...],
                    preferred_element_type=jnp.float32)
        iota = lax.broadcasted_iota(jnp.int32, (blk, _E), 1)
        big = jnp.int32(1 << 30)
        m1 = jnp.max(l, axis=1, keepdims=True)
        i1 = jnp.min(jnp.where(l == m1, iota, big), axis=1, keepdims=True)
        l2 = jnp.where(iota == i1, -jnp.inf, l)
        m2 = jnp.max(l2, axis=1, keepdims=True)
        i2 = jnp.min(jnp.where(l2 == m2, iota, big), axis=1, keepdims=True)
        w1 = jax.nn.sigmoid(m1 - m2)
        idx_ref[...] = jnp.concatenate([i1, i2], axis=1)
        w_ref[...] = jnp.concatenate([w1, 1.0 - w1], axis=1)

    return pl.pallas_call(
        body,
        grid=(_N // blk,),
        in_specs=[
            pl.BlockSpec((blk, _D), lambda i: (i, 0)),
            pl.BlockSpec((_D, _E), lambda i: (0, 0)),
        ],
        out_specs=[
            pl.BlockSpec((blk, _K), lambda i: (i, 0)),
            pl.BlockSpec((blk, _K), lambda i: (i, 0)),
        ],
        out_shape=[
            jax.ShapeDtypeStruct((_N, _K), jnp.int32),
            jax.ShapeDtypeStruct((_N, _K), jnp.float32),
        ],
    )(x_flat, router)


def _sc_gather_rows(table, idx):
    b = idx.shape[0]
    b_per_w = b // _SC_WORKERS
    n_chunks = b_per_w // _SC_CHUNK
    mesh = plsc.VectorSubcoreMesh(core_axis_name="c", subcore_axis_name="s")

    @functools.partial(
        pl.kernel,
        mesh=mesh,
        out_type=jax.ShapeDtypeStruct((b, _D), jnp.float32),
        scratch_types=[
            pltpu.VMEM((b_per_w,), jnp.int32),
            pltpu.VMEM((_SC_CHUNK, _D), jnp.float32),
            pltpu.SemaphoreType.DMA,
        ],
    )
    def k(table_hbm, idx_hbm, out_hbm, idx_v, rows_v, sem):
        wid = lax.axis_index("s") * _SC_CORES + lax.axis_index("c")
        base = wid * b_per_w
        pltpu.sync_copy(idx_hbm.at[pl.ds(base, b_per_w)], idx_v)
        for c in range(n_chunks):
            pltpu.async_copy(
                table_hbm.at[idx_v.at[pl.ds(c * _SC_CHUNK, _SC_CHUNK)]],
                rows_v, sem).wait()
            pltpu.sync_copy(rows_v,
                            out_hbm.at[pl.ds(base + c * _SC_CHUNK, _SC_CHUNK)])

    return k(table, idx)


def _grouped_mlp(e_block, valid, xd, w_pad, w_up_gate, w_down):
    def body(eb_ref, vb_ref, xd_ref, w_ref, wug_ref, wd_ref, o_ref):
        j = pl.program_id(0)

        @pl.when(vb_ref[j] > 0)
        def _():
            xb = xd_ref[0].astype(jnp.bfloat16)
            wug = wug_ref[0].astype(jnp.bfloat16)
            ug = jnp.dot(xb, wug, preferred_element_type=jnp.float32)
            gate = ug[:, :_I]
            up = ug[:, _I:]
            h = (gate * jax.nn.sigmoid(gate)) * up
            y = jnp.dot(h.astype(jnp.bfloat16),
                        wd_ref[0].astype(jnp.bfloat16),
                        preferred_element_type=jnp.float32)
            o_ref[0] = y * w_ref[0]

    grid_spec = pltpu.PrefetchScalarGridSpec(
        num_scalar_prefetch=2,
        grid=(_NB,),
        in_specs=[
            pl.BlockSpec((1, _T, _D), lambda j, eb, vb: (j, 0, 0)),
            pl.BlockSpec((1, _T, 1), lambda j, eb, vb: (j, 0, 0)),
            pl.BlockSpec((1, _D, 2 * _I), lambda j, eb, vb: (eb[j], 0, 0)),
            pl.BlockSpec((1, _I, _D), lambda j, eb, vb: (eb[j], 0, 0)),
        ],
        out_specs=pl.BlockSpec((1, _T, _D), lambda j, eb, vb: (j, 0, 0)),
    )
    return pl.pallas_call(
        body,
        grid_spec=grid_spec,
        out_shape=jax.ShapeDtypeStruct((_NB, _T, _D), jnp.float32),
    )(e_block, valid, xd, w_pad, w_up_gate, w_down)


def _pair_add(g):
    blk = 256

    def body(a_ref, b_ref, o_ref):
        o_ref[...] = a_ref[...] + b_ref[...]

    nblk = _N // blk
    return pl.pallas_call(
        body,
        grid=(nblk,),
        in_specs=[
            pl.BlockSpec((blk, _D), lambda i: (i, 0)),
            pl.BlockSpec((blk, _D), lambda i, _n=nblk: (i + _n, 0)),
        ],
        out_specs=pl.BlockSpec((blk, _D), lambda i: (i, 0)),
        out_shape=jax.ShapeDtypeStruct((_N, _D), jnp.float32),
    )(g, g)


def kernel(x, router, w_up_gate, w_down):
    b, s, d = x.shape
    x_flat = x.reshape(_N, _D)

    top_idx, combine = _router_top2(x_flat, router)
    e1 = top_idx[:, 0]
    e2 = top_idx[:, 1]

    oh1 = jax.nn.one_hot(e1, _E, dtype=jnp.int32)
    oh2 = jax.nn.one_hot(e2, _E, dtype=jnp.int32)
    both = oh1 + oh2
    prefix = jnp.cumsum(both, axis=0) - both
    rank1 = jnp.sum(prefix * oh1, axis=1)
    rank2 = jnp.sum(prefix * oh2, axis=1)
    counts = jnp.sum(both, axis=0)
    nb_e = (counts + _T - 1) // _T
    cum_nb = jnp.cumsum(nb_e)
    total_nb = cum_nb[-1]
    padded_off = (cum_nb - nb_e) * _T

    jarr = jnp.arange(_NB, dtype=jnp.int32)
    e_block = jnp.minimum(
        jnp.searchsorted(cum_nb, jarr, side="right"), _E - 1
    ).astype(jnp.int32)
    valid = (jarr < total_nb).astype(jnp.int32)

    pp1 = padded_off[e1] + rank1
    pp2 = padded_off[e2] + rank2
    tok = jnp.arange(_N, dtype=jnp.int32)
    pad_base = jnp.arange(_ND, dtype=jnp.int32) % _N
    tok_pad = pad_base.at[pp1].set(tok).at[pp2].set(tok)
    w_pad = (jnp.zeros((_ND,), jnp.float32)
             .at[pp1].set(combine[:, 0]).at[pp2].set(combine[:, 1]))
    gidx = jnp.concatenate([pp1, pp2])

    xd = _sc_gather_rows(x_flat, tok_pad)

    yd = _grouped_mlp(e_block, valid,
                      xd.reshape(_NB, _T, _D),
                      w_pad.reshape(_NB, _T, 1),
                      w_up_gate, w_down)

    g = _sc_gather_rows(yd.reshape(_ND, _D), gidx)
    out = _pair_add(g)
    return out.reshape(b, s, d)


# speedup vs baseline: 1.0167x; 1.0167x over previous
import functools

import jax
import jax.numpy as jnp
from jax import lax
from jax.experimental import pallas as pl
from jax.experimental.pallas import tpu as pltpu
from jax.experimental.pallas import tpu_sc as plsc

_N = 2048
_D = 1024
_E = 64
_I = 2048
_K = 2
_A = _N * _K
_T = 128
_NB = 96
_ND = _NB * _T

_SC_CORES = 2
_SC_SUBCORES = 16
_SC_WORKERS = _SC_CORES * _SC_SUBCORES
_SC_CHUNK = 64


def _router_top2(x_flat, router):
    blk = 256

    def body(x_ref, r_ref, idx_ref, w_ref):
        l = jnp.dot(x_ref[...], r_ref[...],
                    preferred_element_type=jnp.float32)
        iota = lax.broadcasted_iota(jnp.int32, (blk, _E), 1)
        big = jnp.int32(1 << 30)
        m1 = jnp.max(l, axis=1, keepdims=True)
        i1 = jnp.min(jnp.where(l == m1, iota, big), axis=1, keepdims=True)
        l2 = jnp.where(iota == i1, -jnp.inf, l)
        m2 = jnp.max(l2, axis=1, keepdims=True)
        i2 = jnp.min(jnp.where(l2 == m2, iota, big), axis=1, keepdims=True)
        w1 = jax.nn.sigmoid(m1 - m2)
        idx_ref[...] = jnp.concatenate([i1, i2], axis=1)
        w_ref[...] = jnp.concatenate([w1, 1.0 - w1], axis=1)

    return pl.pallas_call(
        body,
        grid=(_N // blk,),
        in_specs=[
            pl.BlockSpec((blk, _D), lambda i: (i, 0)),
            pl.BlockSpec((_D, _E), lambda i: (0, 0)),
        ],
        out_specs=[
            pl.BlockSpec((blk, _K), lambda i: (i, 0)),
            pl.BlockSpec((blk, _K), lambda i: (i, 0)),
        ],
        out_shape=[
            jax.ShapeDtypeStruct((_N, _K), jnp.int32),
            jax.ShapeDtypeStruct((_N, _K), jnp.float32),
        ],
    )(x_flat, router)


def _rank_assignments(top_idx):
    blk = 256

    def body(idx_ref, ranks_ref, counts_ref, carry_ref):
        step = pl.program_id(0)

        @pl.when(step == 0)
        def _():
            carry_ref[...] = jnp.zeros_like(carry_ref)

        e = idx_ref[...]
        lane = lax.broadcasted_iota(jnp.int32, (blk, _E), 1)
        oh1 = (lane == e[:, 0:1]).astype(jnp.float32)
        oh2 = (lane == e[:, 1:2]).astype(jnp.float32)
        both = oh1 + oh2
        row = lax.broadcasted_iota(jnp.int32, (blk, blk), 0)
        col = lax.broadcasted_iota(jnp.int32, (blk, blk), 1)
        ltri = (col < row).astype(jnp.float32)
        pre = jnp.dot(ltri, both, preferred_element_type=jnp.float32)
        pre = pre + carry_ref[0:1, :_E]
        r1 = jnp.sum(pre * oh1, axis=1, keepdims=True)
        r2 = jnp.sum(pre * oh2, axis=1, keepdims=True)
        ranks_ref[...] = jnp.concatenate([r1, r2], axis=1).astype(jnp.int32)
        carry_ref[0:1, :_E] = (carry_ref[0:1, :_E]
                               + jnp.sum(both, axis=0, keepdims=True))

        @pl.when(step == _N // blk - 1)
        def _():
            counts_ref[...] = carry_ref[0:1, :_E].astype(jnp.int32)

    return pl.pallas_call(
        body,
        grid=(_N // blk,),
        in_specs=[pl.BlockSpec((blk, _K), lambda i: (i, 0))],
        out_specs=[
            pl.BlockSpec((blk, _K), lambda i: (i, 0)),
            pl.BlockSpec((1, _E), lambda i: (0, 0)),
        ],
        out_shape=[
            jax.ShapeDtypeStruct((_N, _K), jnp.int32),
            jax.ShapeDtypeStruct((1, _E), jnp.int32),
        ],
        scratch_shapes=[pltpu.VMEM((8, 128), jnp.float32)],
    )(top_idx)


def _sc_gather_rows(table, idx):
    b = idx.shape[0]
    b_per_w = b // _SC_WORKERS
    n_chunks = b_per_w // _SC_CHUNK
    mesh = plsc.VectorSubcoreMesh(core_axis_name="c", subcore_axis_name="s")

    @functools.partial(
        pl.kernel,
        mesh=mesh,
        out_type=jax.ShapeDtypeStruct((b, _D), jnp.float32),
        scratch_types=[
            pltpu.VMEM((b_per_w,), jnp.int32),
            pltpu.VMEM((_SC_CHUNK, _D), jnp.float32),
            pltpu.SemaphoreType.DMA,
        ],
    )
    def k(table_hbm, idx_hbm, out_hbm, idx_v, rows_v, sem):
        wid = lax.axis_index("s") * _SC_CORES + lax.axis_index("c")
        base = wid * b_per_w
        pltpu.sync_copy(idx_hbm.at[pl.ds(base, b_per_w)], idx_v)
        for c in range(n_chunks):
            pltpu.async_copy(
                table_hbm.at[idx_v.at[pl.ds(c * _SC_CHUNK, _SC_CHUNK)]],
                rows_v, sem).wait()
            pltpu.sync_copy(rows_v,
                            out_hbm.at[pl.ds(base + c * _SC_CHUNK, _SC_CHUNK)])

    return k(table, idx)


def _grouped_mlp(e_block, valid, xd, w_pad, w_up_gate, w_down):
    def body(eb_ref, vb_ref, xd_ref, w_ref, wug_ref, wd_ref, o_ref):
        j = pl.program_id(0)

        @pl.when(vb_ref[j] > 0)
        def _():
            xb = xd_ref[0].astype(jnp.bfloat16)
            wug = wug_ref[0].astype(jnp.bfloat16)
            ug = jnp.dot(xb, wug, preferred_element_type=jnp.float32)
            gate = ug[:, :_I]
            up = ug[:, _I:]
            h = (gate * jax.nn.sigmoid(gate)) * up
            y = jnp.dot(h.astype(jnp.bfloat16),
                        wd_ref[0].astype(jnp.bfloat16),
                        preferred_element_type=jnp.float32)
            o_ref[0] = y * w_ref[0]

    grid_spec = pltpu.PrefetchScalarGridSpec(
        num_scalar_prefetch=2,
        grid=(_NB,),
        in_specs=[
            pl.BlockSpec((1, _T, _D), lambda j, eb, vb: (j, 0, 0)),
            pl.BlockSpec((1, _T, 1), lambda j, eb, vb: (j, 0, 0)),
            pl.BlockSpec((1, _D, 2 * _I), lambda j, eb, vb: (eb[j], 0, 0)),
            pl.BlockSpec((1, _I, _D), lambda j, eb, vb: (eb[j], 0, 0)),
        ],
        out_specs=pl.BlockSpec((1, _T, _D), lambda j, eb, vb: (j, 0, 0)),
    )
    return pl.pallas_call(
        body,
        grid_spec=grid_spec,
        out_shape=jax.ShapeDtypeStruct((_NB, _T, _D), jnp.float32),
    )(e_block, valid, xd, w_pad, w_up_gate, w_down)


def _pair_add(g):
    blk = 256

    def body(a_ref, b_ref, o_ref):
        o_ref[...] = a_ref[...] + b_ref[...]

    nblk = _N // blk
    return pl.pallas_call(
        body,
        grid=(nblk,),
        in_specs=[
            pl.BlockSpec((blk, _D), lambda i: (i, 0)),
            pl.BlockSpec((blk, _D), lambda i, _n=nblk: (i + _n, 0)),
        ],
        out_specs=pl.BlockSpec((blk, _D), lambda i: (i, 0)),
        out_shape=jax.ShapeDtypeStruct((_N, _D), jnp.float32),
    )(g, g)


def kernel(x, router, w_up_gate, w_down):
    b, s, d = x.shape
    x_flat = x.reshape(_N, _D)

    top_idx, combine = _router_top2(x_flat, router)
    e1 = top_idx[:, 0]
    e2 = top_idx[:, 1]

    ranks, counts2d = _rank_assignments(top_idx)
    counts = counts2d[0]
    nb_e = (counts + _T - 1) // _T
    cum_nb = jnp.cumsum(nb_e)
    total_nb = cum_nb[-1]
    padded_off = (cum_nb - nb_e) * _T

    jarr = jnp.arange(_NB, dtype=jnp.int32)
    e_block = jnp.minimum(
        jnp.searchsorted(cum_nb, jarr, side="right"), _E - 1
    ).astype(jnp.int32)
    valid = (jarr < total_nb).astype(jnp.int32)

    pp1 = padded_off[e1] + ranks[:, 0]
    pp2 = padded_off[e2] + ranks[:, 1]
    tok = jnp.arange(_N, dtype=jnp.int32)
    pad_base = jnp.arange(_ND, dtype=jnp.int32) % _N
    tok_pad = pad_base.at[pp1].set(tok).at[pp2].set(tok)
    w_pad = (jnp.zeros((_ND,), jnp.float32)
             .at[pp1].set(combine[:, 0]).at[pp2].set(combine[:, 1]))
    gidx = jnp.concatenate([pp1, pp2])

    xd = _sc_gather_rows(x_flat, tok_pad)

    yd = _grouped_mlp(e_block, valid,
                      xd.reshape(_NB, _T, _D),
                      w_pad.reshape(_NB, _T, 1),
                      w_up_gate, w_down)

    g = _sc_gather_rows(yd.reshape(_ND, _D), gidx)
    out = _pair_add(g)
    return out.reshape(b, s, d)
